# baseline (speedup 1.0000x reference)
import functools

import jax
import jax.numpy as jnp
import numpy as np
from jax import lax
from jax.experimental import pallas as pl
from jax.experimental.pallas import tpu as pltpu
from jax.experimental.pallas import tpu_sc as plsc

NUM_ELEMENTS = 1000
EMBED_DIM = 256
NUM_BLOCKS = 3
TEMPERATURE = 1.0
B = 4
T = 2048
K = 12


_ROT = [(13, 15, 26, 6), (17, 29, 16, 24)]


def _threefry2x32(k1, k2, x0, x1):
    ks = [np.uint32(k1), np.uint32(k2),
          np.uint32(np.uint32(k1) ^ np.uint32(k2) ^ np.uint32(0x1BD11BDA))]
    x0 = (x0 + ks[0]).astype(np.uint32)
    x1 = (x1 + ks[1]).astype(np.uint32)
    for i in range(5):
        for r in _ROT[i % 2]:
            x0 = (x0 + x1).astype(np.uint32)
            x1 = ((x1 << np.uint32(r)) | (x1 >> np.uint32(32 - r))).astype(np.uint32)
            x1 = (x1 ^ x0).astype(np.uint32)
        x0 = (x0 + ks[(i + 1) % 3]).astype(np.uint32)
        x1 = (x1 + ks[(i + 2) % 3] + np.uint32(i + 1)).astype(np.uint32)
    return x0, x1


def _np_split(key):
    b1, b2 = _threefry2x32(key[0], key[1],
                           np.zeros(2, np.uint32), np.arange(2, dtype=np.uint32))
    return np.stack([b1, b2], axis=1)


def _np_uniform01(key, size):
    iota = np.arange(size, dtype=np.uint64)
    c1 = (iota >> np.uint64(32)).astype(np.uint32)
    c2 = (iota & np.uint64(0xFFFFFFFF)).astype(np.uint32)
    b1, b2 = _threefry2x32(key[0], key[1], c1, c2)
    bits = (b1 ^ b2).astype(np.uint32)
    f = ((bits >> np.uint32(9)) | np.uint32(0x3F800000)).view(np.float32)
    return f - np.float32(1.0)


_NOISE_CACHE = []


def _noise_constant():
    if not _NOISE_CACHE:
        key = np.array([0, 42], dtype=np.uint32)
        blocks = []
        for _ in range(NUM_BLOCKS):
            keys = _np_split(key)
            key, sub = keys[0], keys[1]
            u = _np_uniform01(sub, B * T * NUM_ELEMENTS).reshape(B, T, NUM_ELEMENTS)
            blocks.append(np.float32(1.0) - np.float32(TEMPERATURE) * u)
        _NOISE_CACHE.append(np.stack(blocks))
    return _NOISE_CACHE[0]


def _decoder_body(x_ref, w_ref, noise_ref, idx_ref):
    xb = x_ref[0]
    w = w_ref[0]
    dist = lax.dot_general(xb, w, (((1,), (1,)), ((), ())),
                           preferred_element_type=jnp.float32)
    noisy = dist * noise_ref[0, 0]
    work = noisy
    neg = jnp.float32(-jnp.inf)
    m = jnp.max(work, axis=0)
    for _ in range(K - 1):
        work = jnp.where(work == m[None, :], neg, work)
        m = jnp.max(work, axis=0)
    thr = m
    masked = jnp.where(noisy >= thr[None, :], noisy, 0.0)
    m1 = jnp.max(masked, axis=1)
    lane = lax.broadcasted_iota(jnp.int32, (T, NUM_ELEMENTS), 1)
    big = jnp.int32(2**30)
    idx = jnp.min(jnp.where(masked == m1[:, None], lane, big), axis=1)
    idx_ref[0, 0] = idx


def _routing_call(x, w_stack, noise):
    grid = (NUM_BLOCKS * B,)
    return pl.pallas_call(
        _decoder_body,
        grid=grid,
        in_specs=[
            pl.BlockSpec((1, T, EMBED_DIM), lambda g: (g % B, 0, g // B)),
            pl.BlockSpec((1, NUM_ELEMENTS, EMBED_DIM), lambda g: (g // B, 0, 0)),
            pl.BlockSpec((1, 1, T, NUM_ELEMENTS), lambda g: (g // B, g % B, 0, 0)),
        ],
        out_specs=[
            pl.BlockSpec((1, 1, T), lambda g: (g, 0, 0)),
        ],
        out_shape=[
            jax.ShapeDtypeStruct((NUM_BLOCKS * B, 1, T), jnp.int32),
        ],
        compiler_params=pltpu.CompilerParams(
            vmem_limit_bytes=110 * 1024 * 1024,
        ),
    )(x, w_stack, noise)


def _dist_body(x_ref, wbd_ref, dist_ref):
    dist_ref[0] = lax.dot_general(
        x_ref[0], wbd_ref[...], (((1,), (0,)), ((), ())),
        preferred_element_type=jnp.float32)


_TT = 512


def _dist_call(x, wbd):
    return pl.pallas_call(
        _dist_body,
        grid=(B, T // _TT),
        in_specs=[
            pl.BlockSpec((1, _TT, NUM_BLOCKS * EMBED_DIM), lambda b, t: (b, t, 0)),
            pl.BlockSpec((NUM_BLOCKS * EMBED_DIM, NUM_BLOCKS * NUM_ELEMENTS),
                         lambda b, t: (0, 0)),
        ],
        out_specs=pl.BlockSpec((1, _TT, NUM_BLOCKS * NUM_ELEMENTS),
                               lambda b, t: (b, t, 0)),
        out_shape=jax.ShapeDtypeStruct(
            (B, T, NUM_BLOCKS * NUM_ELEMENTS), jnp.float32),
    )(x, wbd)


def _make_sc_gather(d, n_idx, chunk):
    n_workers = 32
    per_w = n_idx // n_workers
    n_chunks = per_w // chunk
    mesh = plsc.VectorSubcoreMesh(core_axis_name="c", subcore_axis_name="s")

    @functools.partial(
        pl.kernel, mesh=mesh,
        out_type=jax.ShapeDtypeStruct((n_idx, d), jnp.float32),
        scratch_types=[
            pltpu.VMEM((chunk,), jnp.int32),
            pltpu.VMEM((chunk, d), jnp.float32),
            pltpu.SemaphoreType.DMA,
        ],
    )
    def gather_kernel(table_hbm, idx_hbm, out_hbm, idx_v, rows_v, sem):
        wid = lax.axis_index("s") * 2 + lax.axis_index("c")
        base = wid * per_w
        for c in range(n_chunks):
            off = base + c * chunk
            pltpu.sync_copy(idx_hbm.at[pl.ds(off, chunk)], idx_v)
            pltpu.async_copy(table_hbm.at[idx_v], rows_v, sem).wait()
            pltpu.sync_copy(rows_v, out_hbm.at[pl.ds(off, chunk)])

    return gather_kernel


def kernel(x, W0, W1, W2):
    noise = jnp.asarray(_noise_constant())
    w_stack = jnp.stack([W0, W1, W2])

    (idx,) = _routing_call(x, w_stack, noise)

    wbd = jnp.zeros((NUM_BLOCKS, EMBED_DIM, NUM_BLOCKS, NUM_ELEMENTS),
                    jnp.float32)
    for i in range(NUM_BLOCKS):
        wbd = wbd.at[i, :, i, :].set(jnp.transpose(w_stack[i]))
    wbd = wbd.reshape(NUM_BLOCKS * EMBED_DIM, NUM_BLOCKS * NUM_ELEMENTS)
    dist = _dist_call(x, wbd)
    dist_logits = dist.reshape(B, T, NUM_BLOCKS, NUM_ELEMENTS)
    decoded_indices = jnp.transpose(
        idx.reshape(NUM_BLOCKS, B, T), (1, 2, 0))

    table = jnp.concatenate([W0, W1, W2], axis=0)
    gidx = (decoded_indices
            + jnp.arange(NUM_BLOCKS, dtype=jnp.int32) * NUM_ELEMENTS)
    flat_idx = gidx.reshape(-1)
    gather = _make_sc_gather(EMBED_DIM, B * T * NUM_BLOCKS, 128)
    rows = gather(table, flat_idx)
    decoded_latents = rows.reshape(B, T, NUM_BLOCKS * EMBED_DIM)

    return decoded_indices, decoded_latents, dist_logits


# speedup vs baseline: 1.0925x; 1.0925x over previous
import functools

import jax
import jax.numpy as jnp
import numpy as np
from jax import lax
from jax.experimental import pallas as pl
from jax.experimental.pallas import tpu as pltpu
from jax.experimental.pallas import tpu_sc as plsc

NUM_ELEMENTS = 1000
EMBED_DIM = 256
NUM_BLOCKS = 3
TEMPERATURE = 1.0
B = 4
T = 2048
K = 12


_ROT = [(13, 15, 26, 6), (17, 29, 16, 24)]


def _threefry2x32(k1, k2, x0, x1):
    ks = [np.uint32(k1), np.uint32(k2),
          np.uint32(np.uint32(k1) ^ np.uint32(k2) ^ np.uint32(0x1BD11BDA))]
    x0 = (x0 + ks[0]).astype(np.uint32)
    x1 = (x1 + ks[1]).astype(np.uint32)
    for i in range(5):
        for r in _ROT[i % 2]:
            x0 = (x0 + x1).astype(np.uint32)
            x1 = ((x1 << np.uint32(r)) | (x1 >> np.uint32(32 - r))).astype(np.uint32)
            x1 = (x1 ^ x0).astype(np.uint32)
        x0 = (x0 + ks[(i + 1) % 3]).astype(np.uint32)
        x1 = (x1 + ks[(i + 2) % 3] + np.uint32(i + 1)).astype(np.uint32)
    return x0, x1


def _np_split(key):
    b1, b2 = _threefry2x32(key[0], key[1],
                           np.zeros(2, np.uint32), np.arange(2, dtype=np.uint32))
    return np.stack([b1, b2], axis=1)


def _np_uniform01(key, size):
    iota = np.arange(size, dtype=np.uint64)
    c1 = (iota >> np.uint64(32)).astype(np.uint32)
    c2 = (iota & np.uint64(0xFFFFFFFF)).astype(np.uint32)
    b1, b2 = _threefry2x32(key[0], key[1], c1, c2)
    bits = (b1 ^ b2).astype(np.uint32)
    f = ((bits >> np.uint32(9)) | np.uint32(0x3F800000)).view(np.float32)
    return f - np.float32(1.0)


_NOISE_CACHE = []


def _noise_constant():
    if not _NOISE_CACHE:
        key = np.array([0, 42], dtype=np.uint32)
        blocks = []
        for _ in range(NUM_BLOCKS):
            keys = _np_split(key)
            key, sub = keys[0], keys[1]
            u = _np_uniform01(sub, B * T * NUM_ELEMENTS).reshape(B, T, NUM_ELEMENTS)
            blocks.append(np.float32(1.0) - np.float32(TEMPERATURE) * u)
        _NOISE_CACHE.append(np.stack(blocks))
    return _NOISE_CACHE[0]


def _decoder_body(x_ref, w_ref, noise_ref, idx_ref):
    xb = x_ref[0]
    w = w_ref[0]
    dist = lax.dot_general(xb, w, (((1,), (1,)), ((), ())),
                           preferred_element_type=jnp.float32)
    noisy = dist * noise_ref[0, 0]
    work = noisy
    neg = jnp.float32(-jnp.inf)
    m = jnp.max(work, axis=0)
    for _ in range(K - 1):
        work = jnp.where(work == m[None, :], neg, work)
        m = jnp.max(work, axis=0)
    thr = m
    masked = jnp.where(noisy >= thr[None, :], noisy, 0.0)
    m1 = jnp.max(masked, axis=1)
    lane = lax.broadcasted_iota(jnp.int32, (T, NUM_ELEMENTS), 1)
    big = jnp.int32(2**30)
    idx = jnp.min(jnp.where(masked == m1[:, None], lane, big), axis=1)
    idx_ref[0, 0] = idx


def _routing_call(x, w_stack, noise):
    grid = (NUM_BLOCKS * B,)
    return pl.pallas_call(
        _decoder_body,
        grid=grid,
        in_specs=[
            pl.BlockSpec((1, T, EMBED_DIM), lambda g: (g % B, 0, g // B)),
            pl.BlockSpec((1, NUM_ELEMENTS, EMBED_DIM), lambda g: (g // B, 0, 0)),
            pl.BlockSpec((1, 1, T, NUM_ELEMENTS), lambda g: (g // B, g % B, 0, 0)),
        ],
        out_specs=[
            pl.BlockSpec((1, 1, T), lambda g: (g, 0, 0)),
        ],
        out_shape=[
            jax.ShapeDtypeStruct((NUM_BLOCKS * B, 1, T), jnp.int32),
        ],
        compiler_params=pltpu.CompilerParams(
            vmem_limit_bytes=110 * 1024 * 1024,
        ),
    )(x, w_stack, noise)


_TT = 1024


def _dist_body(x_ref, w_ref, dist_ref):
    for i in range(NUM_BLOCKS):
        xb = x_ref[0, :, i * EMBED_DIM:(i + 1) * EMBED_DIM]
        d = lax.dot_general(xb, w_ref[i], (((1,), (1,)), ((), ())),
                            preferred_element_type=jnp.float32)
        dist_ref[0, :, i * NUM_ELEMENTS:(i + 1) * NUM_ELEMENTS] = d


def _dist_call(x, w_stack):
    return pl.pallas_call(
        _dist_body,
        grid=(B, T // _TT),
        in_specs=[
            pl.BlockSpec((1, _TT, NUM_BLOCKS * EMBED_DIM), lambda b, t: (b, t, 0)),
            pl.BlockSpec((NUM_BLOCKS, NUM_ELEMENTS, EMBED_DIM),
                         lambda b, t: (0, 0, 0)),
        ],
        out_specs=pl.BlockSpec((1, _TT, NUM_BLOCKS * NUM_ELEMENTS),
                               lambda b, t: (b, t, 0)),
        out_shape=jax.ShapeDtypeStruct(
            (B, T, NUM_BLOCKS * NUM_ELEMENTS), jnp.float32),
        compiler_params=pltpu.CompilerParams(
            vmem_limit_bytes=60 * 1024 * 1024,
        ),
    )(x, w_stack)


def _make_sc_gather(d, n_idx, chunk):
    n_workers = 32
    per_w = n_idx // n_workers
    n_chunks = per_w // chunk
    mesh = plsc.VectorSubcoreMesh(core_axis_name="c", subcore_axis_name="s")

    @functools.partial(
        pl.kernel, mesh=mesh,
        out_type=jax.ShapeDtypeStruct((n_idx, d), jnp.float32),
        scratch_types=[
            pltpu.VMEM((chunk,), jnp.int32),
            pltpu.VMEM((chunk, d), jnp.float32),
            pltpu.SemaphoreType.DMA,
        ],
    )
    def gather_kernel(table_hbm, idx_hbm, out_hbm, idx_v, rows_v, sem):
        wid = lax.axis_index("s") * 2 + lax.axis_index("c")
        base = wid * per_w
        for c in range(n_chunks):
            off = base + c * chunk
            pltpu.sync_copy(idx_hbm.at[pl.ds(off, chunk)], idx_v)
            pltpu.async_copy(table_hbm.at[idx_v], rows_v, sem).wait()
            pltpu.sync_copy(rows_v, out_hbm.at[pl.ds(off, chunk)])

    return gather_kernel


def kernel(x, W0, W1, W2):
    noise = jnp.asarray(_noise_constant())
    w_stack = jnp.stack([W0, W1, W2])

    (idx,) = _routing_call(x, w_stack, noise)

    dist = _dist_call(x, w_stack)
    dist_logits = dist.reshape(B, T, NUM_BLOCKS, NUM_ELEMENTS)
    decoded_indices = jnp.transpose(
        idx.reshape(NUM_BLOCKS, B, T), (1, 2, 0))

    table = jnp.concatenate([W0, W1, W2], axis=0)
    gidx = (decoded_indices
            + jnp.arange(NUM_BLOCKS, dtype=jnp.int32) * NUM_ELEMENTS)
    flat_idx = gidx.reshape(-1)
    gather = _make_sc_gather(EMBED_DIM, B * T * NUM_BLOCKS, 128)
    rows = gather(table, flat_idx)
    decoded_latents = rows.reshape(B, T, NUM_BLOCKS * EMBED_DIM)

    return decoded_indices, decoded_latents, dist_logits


# speedup vs baseline: 1.2786x; 1.1703x over previous
import functools

import jax
import jax.numpy as jnp
import numpy as np
from jax import lax
from jax.experimental import pallas as pl
from jax.experimental.pallas import tpu as pltpu
from jax.experimental.pallas import tpu_sc as plsc

NUM_ELEMENTS = 1000
EMBED_DIM = 256
NUM_BLOCKS = 3
TEMPERATURE = 1.0
B = 4
T = 2048
K = 12


_ROT = [(13, 15, 26, 6), (17, 29, 16, 24)]


def _threefry2x32(k1, k2, x0, x1):
    ks = [np.uint32(k1), np.uint32(k2),
          np.uint32(np.uint32(k1) ^ np.uint32(k2) ^ np.uint32(0x1BD11BDA))]
    x0 = (x0 + ks[0]).astype(np.uint32)
    x1 = (x1 + ks[1]).astype(np.uint32)
    for i in range(5):
        for r in _ROT[i % 2]:
            x0 = (x0 + x1).astype(np.uint32)
            x1 = ((x1 << np.uint32(r)) | (x1 >> np.uint32(32 - r))).astype(np.uint32)
            x1 = (x1 ^ x0).astype(np.uint32)
        x0 = (x0 + ks[(i + 1) % 3]).astype(np.uint32)
        x1 = (x1 + ks[(i + 2) % 3] + np.uint32(i + 1)).astype(np.uint32)
    return x0, x1


def _np_split(key):
    b1, b2 = _threefry2x32(key[0], key[1],
                           np.zeros(2, np.uint32), np.arange(2, dtype=np.uint32))
    return np.stack([b1, b2], axis=1)


def _np_uniform01(key, size):
    iota = np.arange(size, dtype=np.uint64)
    c1 = (iota >> np.uint64(32)).astype(np.uint32)
    c2 = (iota & np.uint64(0xFFFFFFFF)).astype(np.uint32)
    b1, b2 = _threefry2x32(key[0], key[1], c1, c2)
    bits = (b1 ^ b2).astype(np.uint32)
    f = ((bits >> np.uint32(9)) | np.uint32(0x3F800000)).view(np.float32)
    return f - np.float32(1.0)


_NOISE_CACHE = []


def _noise_constant():
    if not _NOISE_CACHE:
        key = np.array([0, 42], dtype=np.uint32)
        blocks = []
        for _ in range(NUM_BLOCKS):
            keys = _np_split(key)
            key, sub = keys[0], keys[1]
            u = _np_uniform01(sub, B * T * NUM_ELEMENTS).reshape(B, T, NUM_ELEMENTS)
            blocks.append(np.float32(1.0) - np.float32(TEMPERATURE) * u)
        _NOISE_CACHE.append(np.stack(blocks))
    return _NOISE_CACHE[0]


def _decoder_body(x_ref, w_ref, noise_ref, dist_ref, idx_ref):
    xb = x_ref[0]
    w = w_ref[0]
    dist = lax.dot_general(xb, w, (((1,), (1,)), ((), ())),
                           preferred_element_type=jnp.float32)
    dist_ref[0] = dist
    noisy = dist * noise_ref[0, 0]
    neg = jnp.float32(-jnp.inf)
    pos = jnp.float32(jnp.inf)
    work = jnp.maximum(noisy[:T // 2], noisy[T // 2:])
    m = jnp.max(work, axis=0)
    for _ in range(K - 1):
        work = jnp.where(work == m[None, :], neg, work)
        m = jnp.max(work, axis=0)
    cnt0 = jnp.sum((noisy >= m[None, :]).astype(jnp.int32), axis=0)

    def _cond(state):
        _, cnt = state
        return jnp.any(cnt > K)

    def _body(state):
        thr_c, cnt = state
        active = cnt > K
        nxt = jnp.min(jnp.where(noisy > thr_c[None, :], noisy, pos), axis=0)
        new_thr = jnp.where(active, nxt, thr_c)
        new_cnt = jnp.sum((noisy >= new_thr[None, :]).astype(jnp.int32), axis=0)
        return new_thr, jnp.where(active, new_cnt, cnt)

    thr, _ = lax.while_loop(_cond, _body, (m, cnt0))
    masked = jnp.where(noisy >= thr[None, :], noisy, 0.0)
    m1 = jnp.max(masked, axis=1)
    lane = lax.broadcasted_iota(jnp.int32, (T, NUM_ELEMENTS), 1)
    big = jnp.int32(2**30)
    idx = jnp.min(jnp.where(masked == m1[:, None], lane, big), axis=1)
    idx_ref[0, 0] = idx


def _routing_call(x, w_stack, noise):
    grid = (NUM_BLOCKS * B,)
    return pl.pallas_call(
        _decoder_body,
        grid=grid,
        in_specs=[
            pl.BlockSpec((1, T, EMBED_DIM), lambda g: (g % B, 0, g // B)),
            pl.BlockSpec((1, NUM_ELEMENTS, EMBED_DIM), lambda g: (g // B, 0, 0)),
            pl.BlockSpec((1, 1, T, NUM_ELEMENTS), lambda g: (g // B, g % B, 0, 0)),
        ],
        out_specs=[
            pl.BlockSpec((1, T, NUM_ELEMENTS), lambda g: (g, 0, 0)),
            pl.BlockSpec((1, 1, T), lambda g: (g, 0, 0)),
        ],
        out_shape=[
            jax.ShapeDtypeStruct((NUM_BLOCKS * B, T, NUM_ELEMENTS), jnp.float32),
            jax.ShapeDtypeStruct((NUM_BLOCKS * B, 1, T), jnp.int32),
        ],
        compiler_params=pltpu.CompilerParams(
            vmem_limit_bytes=110 * 1024 * 1024,
        ),
    )(x, w_stack, noise)


def _make_sc_gather(d, n_idx, chunk):
    n_workers = 32
    per_w = n_idx // n_workers
    n_chunks = per_w // chunk
    mesh = plsc.VectorSubcoreMesh(core_axis_name="c", subcore_axis_name="s")

    @functools.partial(
        pl.kernel, mesh=mesh,
        out_type=jax.ShapeDtypeStruct((n_idx, d), jnp.float32),
        scratch_types=[
            pltpu.VMEM((chunk,), jnp.int32),
            pltpu.VMEM((chunk, d), jnp.float32),
            pltpu.SemaphoreType.DMA,
        ],
    )
    def gather_kernel(table_hbm, idx_hbm, out_hbm, idx_v, rows_v, sem):
        wid = lax.axis_index("s") * 2 + lax.axis_index("c")
        base = wid * per_w
        for c in range(n_chunks):
            off = base + c * chunk
            pltpu.sync_copy(idx_hbm.at[pl.ds(off, chunk)], idx_v)
            pltpu.async_copy(table_hbm.at[idx_v], rows_v, sem).wait()
            pltpu.sync_copy(rows_v, out_hbm.at[pl.ds(off, chunk)])

    return gather_kernel


def kernel(x, W0, W1, W2):
    noise = jnp.asarray(_noise_constant())
    w_stack = jnp.stack([W0, W1, W2])

    dist, idx = _routing_call(x, w_stack, noise)
    dist_logits = jnp.transpose(
        dist.reshape(NUM_BLOCKS, B, T, NUM_ELEMENTS), (1, 2, 0, 3))
    decoded_indices = jnp.transpose(
        idx.reshape(NUM_BLOCKS, B, T), (1, 2, 0))

    table = jnp.concatenate([W0, W1, W2], axis=0)
    gidx = (decoded_indices
            + jnp.arange(NUM_BLOCKS, dtype=jnp.int32) * NUM_ELEMENTS)
    flat_idx = gidx.reshape(-1)
    gather = _make_sc_gather(EMBED_DIM, B * T * NUM_BLOCKS, 128)
    rows = gather(table, flat_idx)
    decoded_latents = rows.reshape(B, T, NUM_BLOCKS * EMBED_DIM)

    return decoded_indices, decoded_latents, dist_logits
